# Initial kernel scaffold; baseline (speedup 1.0000x reference)
import functools

import jax
import jax.numpy as jnp
from jax import lax
from jax.experimental import pallas as pl
from jax.experimental.pallas import tpu as pltpu
from jax.experimental.pallas import tpu_sc as plsc

N = 10000
E = 320000
D = 128
OUT = 10
EPS = 1e-5

NC = 2
NS = 16
NW = NC * NS
NPAD = 10240
RPT = NPAD // NS
EPW = E // NW
CH = 80
NCHUNK = EPW // CH

RB = 1280
GRID = NPAD // RB

_mesh = plsc.VectorSubcoreMesh(
    core_axis_name="c", subcore_axis_name="s", num_cores=NC, num_subcores=NS
)


@functools.partial(
    pl.kernel,
    out_type=jax.ShapeDtypeStruct((NC, NPAD, 1), jnp.float32),
    mesh=_mesh,
    scratch_types=[
        pltpu.VMEM((NCHUNK, CH), jnp.int32),
        pltpu.VMEM((CH, 1), jnp.float32),
        pltpu.VMEM_SHARED((NPAD, 1), jnp.float32),
    ],
)
def _sc_count(dst_hbm, ones_hbm, zeros_hbm, out_hbm, idx_v, ones_v, acc):
    cid = lax.axis_index("c")
    sid = lax.axis_index("s")
    wid = cid * NS + sid
    pltpu.sync_copy(zeros_hbm, acc.at[pl.ds(sid * RPT, RPT)])
    pltpu.sync_copy(dst_hbm.at[pl.ds(wid * NCHUNK, NCHUNK)], idx_v)
    pltpu.sync_copy(ones_hbm, ones_v)
    plsc.subcore_barrier()

    def body(j, carry):
        pltpu.sync_copy(ones_v, acc.at[idx_v.at[j]], add=True)
        return carry

    lax.fori_loop(0, NCHUNK, body, 0)
    plsc.subcore_barrier()
    pltpu.sync_copy(
        acc.at[pl.ds(sid * RPT, RPT)], out_hbm.at[cid, pl.ds(sid * RPT, RPT)]
    )


@functools.partial(
    pl.kernel,
    out_type=jax.ShapeDtypeStruct((NC, NPAD, D), jnp.float32),
    mesh=_mesh,
    scratch_types=[
        pltpu.VMEM((NCHUNK, CH), jnp.int32),
        pltpu.VMEM((NCHUNK, CH), jnp.int32),
        pltpu.VMEM((CH, D), jnp.float32),
        pltpu.VMEM_SHARED((NPAD, D), jnp.float32),
        pltpu.SemaphoreType.DMA,
    ],
)
def _sc_scatter_rows(
    g_hbm, src_hbm, dst_hbm, zeros_hbm, out_hbm, src_v, dst_v, rows_v, acc, sem
):
    cid = lax.axis_index("c")
    sid = lax.axis_index("s")
    wid = cid * NS + sid
    pltpu.sync_copy(zeros_hbm, acc.at[pl.ds(sid * RPT, RPT)])
    pltpu.sync_copy(src_hbm.at[pl.ds(wid * NCHUNK, NCHUNK)], src_v)
    pltpu.sync_copy(dst_hbm.at[pl.ds(wid * NCHUNK, NCHUNK)], dst_v)
    plsc.subcore_barrier()

    def body(j, carry):
        pltpu.async_copy(g_hbm.at[src_v.at[j]], rows_v, sem).wait()
        pltpu.sync_copy(rows_v, acc.at[dst_v.at[j]], add=True)
        return carry

    lax.fori_loop(0, NCHUNK, body, 0)
    plsc.subcore_barrier()
    pltpu.sync_copy(
        acc.at[pl.ds(sid * RPT, RPT)], out_hbm.at[cid, pl.ds(sid * RPT, RPT)]
    )


def _tc1_body(x_ref, w_ref, cnt_ref, g_ref, h0_ref, dinv_ref, minv_ref):
    h0 = jnp.dot(x_ref[...], w_ref[...], preferred_element_type=jnp.float32)
    p = cnt_ref[0] + cnt_ref[1]
    dinv = lax.rsqrt(p + 1.0)
    h0_ref[...] = h0
    g_ref[...] = h0 * dinv
    dinv_ref[...] = dinv
    minv_ref[...] = 1.0 / jnp.maximum(p, 1.0)


def _tc2_body(s_ref, h0_ref, dinv_ref, pv_ref, h1_ref):
    s = s_ref[0] + s_ref[1]
    dinv = dinv_ref[...]
    agg = dinv * s + (dinv * dinv) * h0_ref[...] + pv_ref[0:1, :]
    a1 = pv_ref[1:2, :] * lax.rsqrt(pv_ref[4:5, :] + EPS)
    h1_ref[...] = jnp.maximum(a1 * (agg - pv_ref[3:4, :]) + pv_ref[2:3, :], 0.0)


def _tc3_body(
    t_ref, h1_ref, minv_ref, wsl_ref, wsr_ref, pv_ref, wl1_ref, wl2_ref,
    bl2_ref, out_ref
):
    mean = (t_ref[0] + t_ref[1]) * minv_ref[...]
    z = (
        jnp.dot(mean, wsl_ref[...], preferred_element_type=jnp.float32)
        + jnp.dot(h1_ref[...], wsr_ref[...], preferred_element_type=jnp.float32)
        + pv_ref[0:1, :]
    )
    a2 = pv_ref[1:2, :] * lax.rsqrt(pv_ref[4:5, :] + EPS)
    h2 = jnp.maximum(a2 * (z - pv_ref[3:4, :]) + pv_ref[2:3, :], 0.0)
    h3 = jnp.maximum(
        jnp.dot(h2, wl1_ref[...], preferred_element_type=jnp.float32)
        + pv_ref[5:6, :],
        0.0,
    )
    logits = (
        jnp.dot(h3, wl2_ref[...], preferred_element_type=jnp.float32)
        + bl2_ref[0:1, :]
    )
    m = jnp.max(logits, axis=1, keepdims=True)
    lse = m + jnp.log(jnp.sum(jnp.exp(logits - m), axis=1, keepdims=True))
    out_ref[...] = logits - lse


def _row_spec(shape):
    return pl.BlockSpec(shape, lambda i: (i, 0))


def _full_spec(shape):
    nd = len(shape)
    return pl.BlockSpec(shape, lambda i, _nd=nd: (0,) * _nd)


_tc1 = pl.pallas_call(
    _tc1_body,
    grid=(GRID,),
    in_specs=[
        _row_spec((RB, D)),
        _full_spec((D, D)),
        pl.BlockSpec((NC, RB, 1), lambda i: (0, i, 0)),
    ],
    out_specs=[
        _row_spec((RB, D)),
        _row_spec((RB, D)),
        _row_spec((RB, 1)),
        _row_spec((RB, 1)),
    ],
    out_shape=[
        jax.ShapeDtypeStruct((NPAD, D), jnp.float32),
        jax.ShapeDtypeStruct((NPAD, D), jnp.float32),
        jax.ShapeDtypeStruct((NPAD, 1), jnp.float32),
        jax.ShapeDtypeStruct((NPAD, 1), jnp.float32),
    ],
)

_tc2 = pl.pallas_call(
    _tc2_body,
    grid=(GRID,),
    in_specs=[
        pl.BlockSpec((NC, RB, D), lambda i: (0, i, 0)),
        _row_spec((RB, D)),
        _row_spec((RB, 1)),
        _full_spec((5, D)),
    ],
    out_specs=_row_spec((RB, D)),
    out_shape=jax.ShapeDtypeStruct((NPAD, D), jnp.float32),
)

_tc3 = pl.pallas_call(
    _tc3_body,
    grid=(GRID,),
    in_specs=[
        pl.BlockSpec((NC, RB, D), lambda i: (0, i, 0)),
        _row_spec((RB, D)),
        _row_spec((RB, 1)),
        _full_spec((D, D)),
        _full_spec((D, D)),
        _full_spec((6, D)),
        _full_spec((D, D)),
        _full_spec((D, D)),
        _full_spec((1, D)),
    ],
    out_specs=_row_spec((RB, D)),
    out_shape=jax.ShapeDtypeStruct((NPAD, D), jnp.float32),
)


def kernel(x, edge_index, batch, W_gcn, b_gcn, bn1_gamma, bn1_beta, bn1_mean,
           bn1_var, W_sage_l, b_sage_l, W_sage_r, bn2_gamma, bn2_beta,
           bn2_mean, bn2_var, W_lin1, b_lin1, W_lin2, b_lin2):
    del batch
    src = edge_index[0].reshape(E // CH, CH)
    dst = edge_index[1].reshape(E // CH, CH)
    ones_col = jnp.ones((CH, 1), jnp.float32)
    zeros_col = jnp.zeros((RPT, 1), jnp.float32)
    zeros_rows = jnp.zeros((RPT, D), jnp.float32)
    xpad = jnp.pad(x, ((0, NPAD - N), (0, 0)))

    cnt = _sc_count(dst, ones_col, zeros_col)

    g, h0, dinv, minv = _tc1(xpad, W_gcn, cnt)

    s_part = _sc_scatter_rows(g, src, dst, zeros_rows)

    pv1 = jnp.stack([b_gcn, bn1_gamma, bn1_beta, bn1_mean, bn1_var])
    h1 = _tc2(s_part, h0, dinv, pv1)

    t_part = _sc_scatter_rows(h1, src, dst, zeros_rows)

    wl1 = jnp.zeros((D, D), jnp.float32).at[:, : D // 2].set(W_lin1)
    bl1 = jnp.zeros((D,), jnp.float32).at[: D // 2].set(b_lin1)
    wl2 = jnp.zeros((D, D), jnp.float32).at[: D // 2, :OUT].set(W_lin2)
    bl2 = jnp.full((1, D), -1e30, jnp.float32).at[0, :OUT].set(b_lin2)
    pv2 = jnp.stack([b_sage_l, bn2_gamma, bn2_beta, bn2_mean, bn2_var, bl1])

    out = _tc3(t_part, h1, minv, W_sage_l, W_sage_r, pv2, wl1, wl2, bl2)
    return out[:N, :OUT]


# speedup vs baseline: 12.9437x; 12.9437x over previous
import functools

import jax
import jax.numpy as jnp
from jax import lax
from jax.experimental import pallas as pl
from jax.experimental.pallas import tpu as pltpu
from jax.experimental.pallas import tpu_sc as plsc

N = 10000
E = 320000
D = 128
OUT = 10
EPS = 1e-5

NC = 2
NS = 16
NW = NC * NS
NPAD = 10240
RPT = NPAD // NS
CH = 128
NCHUNK = 80
EPAD = NW * NCHUNK * CH

RB = 1280
GRID = NPAD // RB

_mesh = plsc.VectorSubcoreMesh(
    core_axis_name="c", subcore_axis_name="s", num_cores=NC, num_subcores=NS
)


def _sc_count_body(
    dst_hbm, ones_hbm, zeros_hbm, out_hbm, idx1_v, ones_v, acc
):
    cid = lax.axis_index("c")
    sid = lax.axis_index("s")
    wid = cid * NS + sid
    pltpu.sync_copy(zeros_hbm, acc.at[pl.ds(sid * RPT, RPT)])
    pltpu.sync_copy(ones_hbm, ones_v)
    plsc.subcore_barrier()

    def body(j, carry):
        pltpu.sync_copy(dst_hbm.at[pl.ds((wid * NCHUNK + j) * CH, CH)], idx1_v)
        pltpu.sync_copy(ones_v, acc.at[idx1_v], add=True)
        return carry

    lax.fori_loop(0, NCHUNK, body, 0)
    plsc.subcore_barrier()
    pltpu.sync_copy(
        acc.at[pl.ds(sid * RPT, RPT)], out_hbm.at[cid, pl.ds(sid * RPT, RPT)]
    )


def _sc_scatter_rows_body(
    g_hbm, src_hbm, dst_hbm, zeros_hbm, out_hbm, src1_v, dst1_v, rows_v,
    acc, sem
):
    cid = lax.axis_index("c")
    sid = lax.axis_index("s")
    wid = cid * NS + sid
    pltpu.sync_copy(zeros_hbm, acc.at[pl.ds(sid * RPT, RPT)])
    plsc.subcore_barrier()

    def body(j, carry):
        base = (wid * NCHUNK + j) * CH
        pltpu.sync_copy(src_hbm.at[pl.ds(base, CH)], src1_v)
        pltpu.sync_copy(dst_hbm.at[pl.ds(base, CH)], dst1_v)
        pltpu.async_copy(g_hbm.at[src1_v], rows_v, sem).wait()
        pltpu.sync_copy(rows_v, acc.at[dst1_v], add=True)
        return carry

    lax.fori_loop(0, NCHUNK, body, 0)
    plsc.subcore_barrier()
    pltpu.sync_copy(
        acc.at[pl.ds(sid * RPT, RPT)], out_hbm.at[cid, pl.ds(sid * RPT, RPT)]
    )


def _make_sc_count(interpret=False):
    return functools.partial(
        pl.kernel,
        out_type=jax.ShapeDtypeStruct((NC, NPAD, 1), jnp.float32),
        mesh=_mesh,
        scratch_types=[
            pltpu.VMEM((CH,), jnp.int32),
            pltpu.VMEM((CH, 1), jnp.float32),
            pltpu.VMEM_SHARED((NPAD, 1), jnp.float32),
        ],
        interpret=interpret,
    )(_sc_count_body)


def _make_sc_scatter_rows(interpret=False):
    return functools.partial(
        pl.kernel,
        out_type=jax.ShapeDtypeStruct((NC, NPAD, D), jnp.float32),
        mesh=_mesh,
        scratch_types=[
            pltpu.VMEM((CH,), jnp.int32),
            pltpu.VMEM((CH,), jnp.int32),
            pltpu.VMEM((CH, D), jnp.float32),
            pltpu.VMEM_SHARED((NPAD, D), jnp.float32),
            pltpu.SemaphoreType.DMA,
        ],
        interpret=interpret,
    )(_sc_scatter_rows_body)


_sc_count = _make_sc_count()
_sc_scatter_rows = _make_sc_scatter_rows()


def _tc1_body(x_ref, w_ref, cnt_ref, g_ref, h0_ref, dinv_ref, minv_ref):
    h0 = jnp.dot(x_ref[...], w_ref[...], preferred_element_type=jnp.float32)
    p = cnt_ref[0] + cnt_ref[1]
    dinv = lax.rsqrt(p + 1.0)
    h0_ref[...] = h0
    g_ref[...] = h0 * dinv
    dinv_ref[...] = dinv
    minv_ref[...] = 1.0 / jnp.maximum(p, 1.0)


def _tc2_body(s_ref, h0_ref, dinv_ref, pv_ref, h1_ref):
    s = s_ref[0] + s_ref[1]
    dinv = dinv_ref[...]
    agg = dinv * s + (dinv * dinv) * h0_ref[...] + pv_ref[0:1, :]
    a1 = pv_ref[1:2, :] * lax.rsqrt(pv_ref[4:5, :] + EPS)
    h1_ref[...] = jnp.maximum(a1 * (agg - pv_ref[3:4, :]) + pv_ref[2:3, :], 0.0)


def _tc3_body(
    t_ref, h1_ref, minv_ref, wsl_ref, wsr_ref, pv_ref, wl1_ref, wl2_ref,
    bl2_ref, out_ref
):
    mean = (t_ref[0] + t_ref[1]) * minv_ref[...]
    z = (
        jnp.dot(mean, wsl_ref[...], preferred_element_type=jnp.float32)
        + jnp.dot(h1_ref[...], wsr_ref[...], preferred_element_type=jnp.float32)
        + pv_ref[0:1, :]
    )
    a2 = pv_ref[1:2, :] * lax.rsqrt(pv_ref[4:5, :] + EPS)
    h2 = jnp.maximum(a2 * (z - pv_ref[3:4, :]) + pv_ref[2:3, :], 0.0)
    h3 = jnp.maximum(
        jnp.dot(h2, wl1_ref[...], preferred_element_type=jnp.float32)
        + pv_ref[5:6, :],
        0.0,
    )
    logits = (
        jnp.dot(h3, wl2_ref[...], preferred_element_type=jnp.float32)
        + bl2_ref[0:1, :]
    )
    m = jnp.max(logits, axis=1, keepdims=True)
    lse = m + jnp.log(jnp.sum(jnp.exp(logits - m), axis=1, keepdims=True))
    out_ref[...] = logits - lse


def _row_spec(shape):
    return pl.BlockSpec(shape, lambda i: (i, 0))


def _full_spec(shape):
    nd = len(shape)
    return pl.BlockSpec(shape, lambda i, _nd=nd: (0,) * _nd)


_tc1 = pl.pallas_call(
    _tc1_body,
    grid=(GRID,),
    in_specs=[
        _row_spec((RB, D)),
        _full_spec((D, D)),
        pl.BlockSpec((NC, RB, 1), lambda i: (0, i, 0)),
    ],
    out_specs=[
        _row_spec((RB, D)),
        _row_spec((RB, D)),
        _row_spec((RB, 1)),
        _row_spec((RB, 1)),
    ],
    out_shape=[
        jax.ShapeDtypeStruct((NPAD, D), jnp.float32),
        jax.ShapeDtypeStruct((NPAD, D), jnp.float32),
        jax.ShapeDtypeStruct((NPAD, 1), jnp.float32),
        jax.ShapeDtypeStruct((NPAD, 1), jnp.float32),
    ],
)

_tc2 = pl.pallas_call(
    _tc2_body,
    grid=(GRID,),
    in_specs=[
        pl.BlockSpec((NC, RB, D), lambda i: (0, i, 0)),
        _row_spec((RB, D)),
        _row_spec((RB, 1)),
        _full_spec((5, D)),
    ],
    out_specs=_row_spec((RB, D)),
    out_shape=jax.ShapeDtypeStruct((NPAD, D), jnp.float32),
)

_tc3 = pl.pallas_call(
    _tc3_body,
    grid=(GRID,),
    in_specs=[
        pl.BlockSpec((NC, RB, D), lambda i: (0, i, 0)),
        _row_spec((RB, D)),
        _row_spec((RB, 1)),
        _full_spec((D, D)),
        _full_spec((D, D)),
        _full_spec((6, D)),
        _full_spec((D, D)),
        _full_spec((D, D)),
        _full_spec((1, D)),
    ],
    out_specs=_row_spec((RB, D)),
    out_shape=jax.ShapeDtypeStruct((NPAD, D), jnp.float32),
)


def kernel(x, edge_index, batch, W_gcn, b_gcn, bn1_gamma, bn1_beta, bn1_mean,
           bn1_var, W_sage_l, b_sage_l, W_sage_r, bn2_gamma, bn2_beta,
           bn2_mean, bn2_var, W_lin1, b_lin1, W_lin2, b_lin2):
    del batch
    pad_i = jnp.arange(EPAD - E, dtype=jnp.int32)
    src = jnp.concatenate([edge_index[0], pad_i % N])
    dst = jnp.concatenate([edge_index[1], N + pad_i % (NPAD - N)])
    ones_col = jnp.ones((CH, 1), jnp.float32)
    zeros_col = jnp.zeros((RPT, 1), jnp.float32)
    zeros_rows = jnp.zeros((RPT, D), jnp.float32)
    xpad = jnp.pad(x, ((0, NPAD - N), (0, 0)))

    cnt = _sc_count(dst, ones_col, zeros_col)

    g, h0, dinv, minv = _tc1(xpad, W_gcn, cnt)

    s_part = _sc_scatter_rows(g, src, dst, zeros_rows)

    pv1 = jnp.stack([b_gcn, bn1_gamma, bn1_beta, bn1_mean, bn1_var])
    h1 = _tc2(s_part, h0, dinv, pv1)

    t_part = _sc_scatter_rows(h1, src, dst, zeros_rows)

    wl1 = jnp.zeros((D, D), jnp.float32).at[:, : D // 2].set(W_lin1)
    bl1 = jnp.zeros((D,), jnp.float32).at[: D // 2].set(b_lin1)
    wl2 = jnp.zeros((D, D), jnp.float32).at[: D // 2, :OUT].set(W_lin2)
    bl2 = jnp.full((1, D), -1e30, jnp.float32).at[0, :OUT].set(b_lin2)
    pv2 = jnp.stack([b_sage_l, bn2_gamma, bn2_beta, bn2_mean, bn2_var, bl1])

    out = _tc3(t_part, h1, minv, W_sage_l, W_sage_r, pv2, wl1, wl2, bl2)
    return out[:N, :OUT]
